# baseline (speedup 1.0000x reference)
import functools

import jax
import jax.numpy as jnp
from jax import lax
from jax.experimental import pallas as pl
from jax.experimental.pallas import tpu as pltpu
from jax.experimental.pallas import tpu_sc as plsc

NE = 64
NG = 8
GSZ = NE // NG
TG = 4
TK = 8
SCALE = 2.5
HID = 4096
TOKENS = 16384
BT = 1024

NW = 32
TPW = TOKENS // NW
LANES = 16
NCHUNK = TPW // LANES


def _score_block(x_ref, w_ref, s_ref):
    logits = jax.lax.dot_general(
        w_ref[...], x_ref[...],
        dimension_numbers=(((1,), (1,)), ((), ())),
        preferred_element_type=jnp.float32,
    )
    s = 1.0 / (1.0 + jnp.exp(-logits))
    for w in range(BT // TPW):
        s_ref[w] = s[:, w * TPW:(w + 1) * TPW]


def _scores_tc(x, weight):
    return pl.pallas_call(
        _score_block,
        grid=(TOKENS // BT,),
        in_specs=[
            pl.BlockSpec((BT, HID), lambda i: (i, 0)),
            pl.BlockSpec((NE, HID), lambda i: (0, 0)),
        ],
        out_specs=pl.BlockSpec((BT // TPW, NE, TPW), lambda i: (i, 0, 0)),
        out_shape=jax.ShapeDtypeStruct((NW, NE, TPW), jnp.float32),
    )(x, weight)


def _route_sc(s3, bias_mat):
    mesh = plsc.VectorSubcoreMesh(core_axis_name="c", subcore_axis_name="s")

    @functools.partial(
        pl.kernel, mesh=mesh,
        out_type=[
            jax.ShapeDtypeStruct((NW, TK, TPW), jnp.int32),
            jax.ShapeDtypeStruct((NW, TK, TPW), jnp.float32),
        ],
        scratch_types=[
            pltpu.VMEM((NE, TPW), jnp.float32),
            pltpu.VMEM((NE, LANES), jnp.float32),
            pltpu.VMEM((TK, TPW), jnp.int32),
            pltpu.VMEM((TK, TPW), jnp.float32),
        ],
    )
    def k(s_hbm, b_hbm, idx_hbm, w_hbm, s_v, b_v, idx_v, w_v):
        wid = lax.axis_index("s") * 2 + lax.axis_index("c")
        pltpu.sync_copy(s_hbm.at[wid], s_v)
        pltpu.sync_copy(b_hbm, b_v)

        def chunk(c, carry):
            off = pl.multiple_of(c * LANES, LANES)
            sr = [s_v[e, pl.ds(off, LANES)] for e in range(NE)]
            sb = [sr[e] + b_v[e, :] for e in range(NE)]

            gs = []
            for g in range(NG):
                m1 = sb[GSZ * g]
                m2 = jnp.full((LANES,), -jnp.inf, jnp.float32)
                for j in range(1, GSZ):
                    v = sb[GSZ * g + j]
                    m2 = jnp.maximum(m2, jnp.minimum(m1, v))
                    m1 = jnp.maximum(m1, v)
                gs.append(m1 + m2)

            rank = [jnp.zeros((LANES,), jnp.float32) for _ in range(NG)]
            for i in range(NG):
                for j in range(i + 1, NG):
                    d = gs[i] >= gs[j]
                    rank[j] = rank[j] + jnp.where(d, 1.0, 0.0)
                    rank[i] = rank[i] + jnp.where(d, 0.0, 1.0)
            gmask = [rank[g] < float(TG) for g in range(NG)]

            ms = [jnp.where(gmask[e // GSZ], sb[e], 0.0) for e in range(NE)]

            iks = []
            ws = []
            for _ in range(TK):
                m = ms[0]
                for e in range(1, NE):
                    m = jnp.maximum(m, ms[e])
                cand = jnp.full((LANES,), NE, jnp.int32)
                for e in range(NE):
                    cand = jnp.minimum(cand, jnp.where(ms[e] == m, e, NE))
                iks.append(cand)
                wk = jnp.zeros((LANES,), jnp.float32)
                nms = []
                for e in range(NE):
                    hit = cand == e
                    wk = jnp.where(hit, sr[e], wk)
                    nms.append(jnp.where(hit, -jnp.inf, ms[e]))
                ms = nms
                ws.append(wk)

            denom = ws[0]
            for t in range(1, TK):
                denom = denom + ws[t]
            denom = denom + 1e-20
            for t in range(TK):
                idx_v[t, pl.ds(off, LANES)] = iks[t]
                w_v[t, pl.ds(off, LANES)] = ws[t] / denom * SCALE
            return carry

        lax.fori_loop(0, NCHUNK, chunk, 0)
        pltpu.sync_copy(idx_v, idx_hbm.at[wid])
        pltpu.sync_copy(w_v, w_hbm.at[wid])

    return k(s3, bias_mat)


def kernel(x, weight, e_score_correction_bias):
    s3 = _scores_tc(x.astype(jnp.float32), weight.astype(jnp.float32))
    bias_mat = jnp.broadcast_to(
        e_score_correction_bias.astype(jnp.float32).reshape(NE, 1), (NE, LANES))
    idx3, w3 = _route_sc(s3, bias_mat)
    idx = idx3.transpose(0, 2, 1).reshape(TOKENS, TK)
    w = w3.transpose(0, 2, 1).reshape(TOKENS, TK)
    return idx, w
